# Initial kernel scaffold; baseline (speedup 1.0000x reference)
import jax
import jax.numpy as jnp
from jax.experimental import pallas as pl


def kernel(x, embedding):
    raise NotImplementedError("write your pallas kernel here")


# speedup vs baseline: 1.1155x; 1.1155x over previous
import functools

import jax
import jax.numpy as jnp
from jax import lax
from jax.experimental import pallas as pl
from jax.experimental.pallas import tpu as pltpu
from jax.experimental.pallas import tpu_sc as plsc

VOCAB = 76
SEQ = 2048
DMODEL = 512
BATCH = 64

NCORES = 2
NSUB = 16
NW = NCORES * NSUB
TOK = BATCH * SEQ
TPW = TOK // NW
CHUNK = 64
NCHUNK = TPW // CHUNK
NLANE = 16


def _pe_body(o_ref):
    r = lax.broadcasted_iota(jnp.int32, (SEQ, DMODEL), 0).astype(jnp.float32)
    c = lax.broadcasted_iota(jnp.int32, (SEQ, DMODEL), 1)
    even = (c - lax.rem(c, 2)).astype(jnp.float32)
    denom = jnp.exp(even * (jnp.log(10000.0) / DMODEL))
    theta = r / denom
    o_ref[...] = jnp.where(lax.rem(c, 2) == 0, jnp.sin(theta), jnp.cos(theta))


_pe_table = pl.pallas_call(
    _pe_body,
    out_shape=jax.ShapeDtypeStruct((SEQ, DMODEL), jnp.float32),
)


def _sc_body(idx_hbm, table_hbm, pe_hbm, out_hbm, idx_v, pe_v, rows_v, sem):
    wid = lax.axis_index("s") * NCORES + lax.axis_index("c")
    base = wid * TPW
    pltpu.sync_copy(idx_hbm.at[wid], idx_v)

    def chunk_body(ci, carry):
        tok0 = base + ci * CHUNK
        s0 = lax.rem(tok0, SEQ)
        pltpu.sync_copy(pe_hbm.at[pl.ds(s0, CHUNK)], pe_v)
        pltpu.async_copy(table_hbm.at[idx_v.at[ci]], rows_v, sem).wait()

        def row_body(i, c2):
            for j in range(DMODEL // NLANE):
                sl = pl.ds(j * NLANE, NLANE)
                rows_v[i, sl] = rows_v[i, sl] + pe_v[i, sl]
            return c2

        lax.fori_loop(0, CHUNK, row_body, 0)
        pltpu.sync_copy(rows_v, out_hbm.at[pl.ds(tok0, CHUNK)])
        return carry

    lax.fori_loop(0, NCHUNK, chunk_body, 0)


_sc_embed = pl.kernel(
    _sc_body,
    out_type=jax.ShapeDtypeStruct((TOK, DMODEL), jnp.float32),
    mesh=plsc.VectorSubcoreMesh(core_axis_name="c", subcore_axis_name="s",
                                num_cores=NCORES, num_subcores=NSUB),
    scratch_types=[
        pltpu.VMEM((NCHUNK, CHUNK), jnp.int32),
        pltpu.VMEM((CHUNK, DMODEL), jnp.float32),
        pltpu.VMEM((CHUNK, DMODEL), jnp.float32),
        pltpu.SemaphoreType.DMA,
    ],
)


def kernel(x, embedding):
    idx = x.reshape(NW, NCHUNK, CHUNK).astype(jnp.int32)
    pe = _pe_table()
    out = _sc_embed(idx, embedding, pe)
    return out.reshape(BATCH, SEQ, DMODEL)
